# Initial kernel scaffold; baseline (speedup 1.0000x reference)
import jax
import jax.numpy as jnp
from jax.experimental import pallas as pl


def kernel(img, lbp_weight, kernel_weight):
    raise NotImplementedError("write your pallas kernel here")


# speedup vs baseline: 12.5285x; 12.5285x over previous
import functools

import jax
import jax.numpy as jnp
from jax import lax
from jax.experimental import pallas as pl
from jax.experimental.pallas import tpu as pltpu
from jax.experimental.pallas import tpu_sc as plsc

_OFFS = [(-1, 1), (0, 1), (1, 1), (1, 0), (1, -1), (0, -1), (-1, -1), (-1, 0)]

_B, _H, _W = 8, 512, 512
_NPIX = _B * _H * _W

_NC, _NS, _L = 2, 16, 16
_NW = _NC * _NS
_CHUNK = _NPIX // _NW
_HBINS = 256
_HSIZE = _L * _HBINS


def _codes_body(img_ref, codes_ref, pad_ref):
    r = img_ref[0, 0]
    g = img_ref[0, 1]
    b = img_ref[0, 2]
    gray = 0.299 * r + 0.587 * g + 0.114 * b
    grayb = gray.astype(jnp.bfloat16).astype(jnp.float32)
    pad_ref[...] = jnp.zeros((_H + 2, _W + 2), jnp.float32)
    pad_ref[1:_H + 1, 1:_W + 1] = grayb
    code = jnp.zeros((_H, _W), jnp.int32)
    for i, (dr, dc) in enumerate(_OFFS):
        nb = pad_ref[1 + dr:_H + 1 + dr, 1 + dc:_W + 1 + dc]
        code = code + jnp.where(nb >= grayb, jnp.int32(1 << i), jnp.int32(0))
    codes_ref[0] = code


def _compute_codes(img):
    return pl.pallas_call(
        _codes_body,
        grid=(_B,),
        in_specs=[pl.BlockSpec((1, 3, _H, _W), lambda b: (b, 0, 0, 0))],
        out_specs=pl.BlockSpec((1, _H, _W), lambda b: (b, 0, 0)),
        out_shape=jax.ShapeDtypeStruct((_B, _H, _W), jnp.int32),
        scratch_shapes=[pltpu.VMEM((_H + 2, _W + 2), jnp.float32)],
    )(img)


def _sc_hist_body(codes_hbm, out_hbm, codes_v, hist_v, sem):
    wid = lax.axis_index("s") * _NC + lax.axis_index("c")
    base = wid * _CHUNK
    pltpu.sync_copy(codes_hbm.at[pl.ds(base, _CHUNK)], codes_v)

    zero = jnp.zeros((_L,), jnp.float32)

    def zbody(i, carry):
        hist_v[pl.ds(i * _L, _L)] = zero
        return carry

    lax.fori_loop(0, _HSIZE // _L, zbody, 0)

    lane_off = lax.iota(jnp.int32, _L) * _HBINS
    ones = jnp.ones((_L,), jnp.float32)

    def body(i, carry):
        c16 = codes_v[pl.ds(i * _L, _L)]
        plsc.addupdate_scatter(hist_v, [c16 + lane_off], ones)
        return carry

    lax.fori_loop(0, _CHUNK // _L, body, 0)
    pltpu.sync_copy(hist_v, out_hbm.at[wid])


@functools.cache
def _sc_hist():
    return pl.kernel(
        _sc_hist_body,
        out_type=jax.ShapeDtypeStruct((_NW, _HSIZE), jnp.float32),
        mesh=plsc.VectorSubcoreMesh(
            core_axis_name="c", subcore_axis_name="s",
            num_cores=_NC, num_subcores=_NS,
        ),
        scratch_types=[
            pltpu.VMEM((_CHUNK,), jnp.int32),
            pltpu.VMEM((_HSIZE,), jnp.float32),
            pltpu.SemaphoreType.DMA,
        ],
        compiler_params=pltpu.CompilerParams(needs_layout_passes=False),
    )


def _finalize_body(parts_ref, out_ref):
    counts = jnp.sum(parts_ref[...], axis=0, keepdims=True)
    mean = jnp.mean(counts)
    var = jnp.sum((counts - mean) ** 2) / jnp.float32(_HBINS - 1)
    out_ref[...] = (counts - mean) * lax.rsqrt(var)


def _finalize(parts):
    return pl.pallas_call(
        _finalize_body,
        out_shape=jax.ShapeDtypeStruct((1, _HBINS), jnp.float32),
    )(parts)


@jax.jit
def kernel(img, lbp_weight, kernel_weight):
    codes = _compute_codes(img)
    parts = _sc_hist()(codes.reshape(_NW * _CHUNK))
    return _finalize(parts.reshape(_NW * _L, _HBINS))


# speedup vs baseline: 14.0611x; 1.1223x over previous
import functools

import jax
import jax.numpy as jnp
from jax import lax
from jax.experimental import pallas as pl
from jax.experimental.pallas import tpu as pltpu
from jax.experimental.pallas import tpu_sc as plsc

_OFFS = [(-1, 1), (0, 1), (1, 1), (1, 0), (1, -1), (0, -1), (-1, -1), (-1, 0)]

_B, _H, _W = 8, 512, 512
_NPIX = _B * _H * _W

_NC, _NS, _L = 2, 16, 16
_NW = _NC * _NS
_CHUNK = _NPIX // _NW
_HBINS = 256
_HSIZE = _L * _HBINS


def _codes_body(img_ref, codes_ref, pad_ref):
    r = img_ref[0, 0]
    g = img_ref[0, 1]
    b = img_ref[0, 2]
    gray = 0.299 * r + 0.587 * g + 0.114 * b
    grayb = gray.astype(jnp.bfloat16).astype(jnp.float32)
    pad_ref[...] = jnp.zeros((_H + 2, _W + 2), jnp.float32)
    pad_ref[1:_H + 1, 1:_W + 1] = grayb
    code = jnp.zeros((_H, _W), jnp.int32)
    for i, (dr, dc) in enumerate(_OFFS):
        nb = pad_ref[1 + dr:_H + 1 + dr, 1 + dc:_W + 1 + dc]
        code = code + jnp.where(nb >= grayb, jnp.int32(1 << i), jnp.int32(0))
    codes_ref[0] = code


def _compute_codes(img):
    return pl.pallas_call(
        _codes_body,
        grid=(_B,),
        in_specs=[pl.BlockSpec((1, 3, _H, _W), lambda b: (b, 0, 0, 0))],
        out_specs=pl.BlockSpec((1, _H, _W), lambda b: (b, 0, 0)),
        out_shape=jax.ShapeDtypeStruct((_B, _H, _W), jnp.int32),
        scratch_shapes=[pltpu.VMEM((_H + 2, _W + 2), jnp.float32)],
    )(img)


_ROWS_PER_TILE = _H // 4


def _sc_hist_body(codes_hbm, out_hbm, codes_v, h_a, h_b, sem0, sem1):
    wid = lax.axis_index("s") * _NC + lax.axis_index("c")
    b = wid // 4
    r0 = (wid % 4) * _ROWS_PER_TILE
    half = _ROWS_PER_TILE // 2
    cp0 = pltpu.async_copy(
        codes_hbm.at[b, pl.ds(r0, half), :], codes_v.at[pl.ds(0, half), :], sem0)
    cp1 = pltpu.async_copy(
        codes_hbm.at[b, pl.ds(r0 + half, half), :],
        codes_v.at[pl.ds(half, half), :], sem1)

    zero = jnp.zeros((_L,), jnp.float32)

    def zbody(i, carry):
        h_a[pl.ds(i * _L, _L)] = zero
        h_b[pl.ds(i * _L, _L)] = zero
        return carry

    lax.fori_loop(0, _HSIZE // _L, zbody, 0)

    lane_off = lax.iota(jnp.int32, _L) * _HBINS
    ones = jnp.ones((_L,), jnp.float32)

    def row_body(r, carry):
        for j in range(_W // _L):
            c16 = codes_v[r, pl.ds(j * _L, _L)]
            tgt = h_a if j % 2 == 0 else h_b
            plsc.addupdate_scatter(tgt, [c16 + lane_off], ones)
        return carry

    cp0.wait()
    lax.fori_loop(0, half, row_body, 0)
    cp1.wait()
    lax.fori_loop(half, _ROWS_PER_TILE, row_body, 0)

    def mbody(i, carry):
        sl = pl.ds(i * _L, _L)
        h_a[sl] = h_a[sl] + h_b[sl]
        return carry

    lax.fori_loop(0, _HSIZE // _L, mbody, 0)
    pltpu.sync_copy(h_a, out_hbm.at[wid])


@functools.cache
def _sc_hist():
    return pl.kernel(
        _sc_hist_body,
        out_type=jax.ShapeDtypeStruct((_NW, _HSIZE), jnp.float32),
        mesh=plsc.VectorSubcoreMesh(
            core_axis_name="c", subcore_axis_name="s",
            num_cores=_NC, num_subcores=_NS,
        ),
        scratch_types=[
            pltpu.VMEM((_ROWS_PER_TILE, _W), jnp.int32),
            pltpu.VMEM((_HSIZE,), jnp.float32),
            pltpu.VMEM((_HSIZE,), jnp.float32),
            pltpu.SemaphoreType.DMA,
            pltpu.SemaphoreType.DMA,
        ],
        compiler_params=pltpu.CompilerParams(needs_layout_passes=False),
    )


def _finalize_body(parts_ref, out_ref):
    counts = jnp.sum(parts_ref[...], axis=0, keepdims=True)
    mean = jnp.mean(counts)
    var = jnp.sum((counts - mean) ** 2) / jnp.float32(_HBINS - 1)
    out_ref[...] = (counts - mean) * lax.rsqrt(var)


def _finalize(parts):
    return pl.pallas_call(
        _finalize_body,
        out_shape=jax.ShapeDtypeStruct((1, _HBINS), jnp.float32),
    )(parts)


@jax.jit
def kernel(img, lbp_weight, kernel_weight):
    codes = _compute_codes(img)
    parts = _sc_hist()(codes)
    return _finalize(parts.reshape(_NW * _L, _HBINS))


# speedup vs baseline: 14.6844x; 1.0443x over previous
import functools

import jax
import jax.numpy as jnp
from jax import lax
from jax.experimental import pallas as pl
from jax.experimental.pallas import tpu as pltpu
from jax.experimental.pallas import tpu_sc as plsc

_OFFS = [(-1, 1), (0, 1), (1, 1), (1, 0), (1, -1), (0, -1), (-1, -1), (-1, 0)]

_B, _H, _W = 8, 512, 512
_NPIX = _B * _H * _W

_NC, _NS, _L = 2, 16, 16
_NW = _NC * _NS
_CHUNK = _NPIX // _NW
_HBINS = 256
_HSIZE = _L * _HBINS


def _codes_body(img_ref, codes_ref, pad_ref):
    r = img_ref[0, 0]
    g = img_ref[0, 1]
    b = img_ref[0, 2]
    gray = 0.299 * r + 0.587 * g + 0.114 * b
    grayb = gray.astype(jnp.bfloat16).astype(jnp.float32)
    pad_ref[...] = jnp.zeros((_H + 2, _W + 2), jnp.float32)
    pad_ref[1:_H + 1, 1:_W + 1] = grayb
    code = jnp.zeros((_H, _W), jnp.int32)
    for i, (dr, dc) in enumerate(_OFFS):
        nb = pad_ref[1 + dr:_H + 1 + dr, 1 + dc:_W + 1 + dc]
        code = code + jnp.where(nb >= grayb, jnp.int32(1 << i), jnp.int32(0))
    col = lax.broadcasted_iota(jnp.int32, (_H, _W), 1)
    codes_ref[0] = code + ((col & (_L - 1)) << 8)


def _compute_codes(img):
    return pl.pallas_call(
        _codes_body,
        grid=(_B,),
        in_specs=[pl.BlockSpec((1, 3, _H, _W), lambda b: (b, 0, 0, 0))],
        out_specs=pl.BlockSpec((1, _H, _W), lambda b: (b, 0, 0)),
        out_shape=jax.ShapeDtypeStruct((_B, _H, _W), jnp.int32),
        scratch_shapes=[pltpu.VMEM((_H + 2, _W + 2), jnp.float32)],
    )(img)


_ROWS_PER_TILE = _H // 4


def _sc_hist_body(codes_hbm, out_hbm, codes_v, h_a, h_b, h_c, h_d, sem0, sem1):
    wid = lax.axis_index("s") * _NC + lax.axis_index("c")
    b = wid // 4
    r0 = (wid % 4) * _ROWS_PER_TILE
    half = _ROWS_PER_TILE // 2
    cp0 = pltpu.async_copy(
        codes_hbm.at[b, pl.ds(r0, half), :], codes_v.at[pl.ds(0, half), :], sem0)
    cp1 = pltpu.async_copy(
        codes_hbm.at[b, pl.ds(r0 + half, half), :],
        codes_v.at[pl.ds(half, half), :], sem1)

    zero = jnp.zeros((_L,), jnp.float32)
    hists = [h_a, h_b, h_c, h_d]

    def zbody(i, carry):
        sl = pl.ds(i * _L, _L)
        for h in hists:
            h[sl] = zero
        return carry

    lax.fori_loop(0, _HSIZE // _L, zbody, 0)

    ones = jnp.ones((_L,), jnp.float32)

    def row_body(r, carry):
        for j in range(_W // _L):
            c16 = codes_v[r, pl.ds(j * _L, _L)]
            plsc.addupdate_scatter(hists[j % 4], [c16], ones)
        return carry

    cp0.wait()
    lax.fori_loop(0, half, row_body, 0)
    cp1.wait()
    lax.fori_loop(half, _ROWS_PER_TILE, row_body, 0)

    def mbody(i, carry):
        sl = pl.ds(i * _L, _L)
        h_a[sl] = (h_a[sl] + h_b[sl]) + (h_c[sl] + h_d[sl])
        return carry

    lax.fori_loop(0, _HSIZE // _L, mbody, 0)
    pltpu.sync_copy(h_a, out_hbm.at[wid])


@functools.cache
def _sc_hist():
    return pl.kernel(
        _sc_hist_body,
        out_type=jax.ShapeDtypeStruct((_NW, _HSIZE), jnp.float32),
        mesh=plsc.VectorSubcoreMesh(
            core_axis_name="c", subcore_axis_name="s",
            num_cores=_NC, num_subcores=_NS,
        ),
        scratch_types=[
            pltpu.VMEM((_ROWS_PER_TILE, _W), jnp.int32),
            pltpu.VMEM((_HSIZE,), jnp.float32),
            pltpu.VMEM((_HSIZE,), jnp.float32),
            pltpu.VMEM((_HSIZE,), jnp.float32),
            pltpu.VMEM((_HSIZE,), jnp.float32),
            pltpu.SemaphoreType.DMA,
            pltpu.SemaphoreType.DMA,
        ],
        compiler_params=pltpu.CompilerParams(needs_layout_passes=False),
    )


def _finalize_body(parts_ref, out_ref):
    counts = jnp.sum(parts_ref[...], axis=0, keepdims=True)
    mean = jnp.mean(counts)
    var = jnp.sum((counts - mean) ** 2) / jnp.float32(_HBINS - 1)
    out_ref[...] = (counts - mean) * lax.rsqrt(var)


def _finalize(parts):
    return pl.pallas_call(
        _finalize_body,
        out_shape=jax.ShapeDtypeStruct((1, _HBINS), jnp.float32),
    )(parts)


@jax.jit
def kernel(img, lbp_weight, kernel_weight):
    codes = _compute_codes(img)
    parts = _sc_hist()(codes)
    return _finalize(parts.reshape(_NW * _L, _HBINS))
